# Initial kernel scaffold; baseline (speedup 1.0000x reference)
import jax
import jax.numpy as jnp
from jax.experimental import pallas as pl


def kernel(input, weight):
    raise NotImplementedError("write your pallas kernel here")


# speedup vs baseline: 1.3927x; 1.3927x over previous
import functools

import jax
import jax.numpy as jnp
from jax import lax
from jax.experimental import pallas as pl
from jax.experimental.pallas import tpu as pltpu
from jax.experimental.pallas import tpu_sc as plsc

K = 8192
D = 256
B, HW = 32, 576
M = B * HW

BM = 512
BK = 512
G = M // BM
NK = K // BK

NC, NS = 2, 16
NW = NC * NS
ROWS_PER_W = M // NW
GCH = 96
NCH = ROWS_PER_W // GCH


def _argmin_body(x_ref, w_ref, idx_ref):
    x = x_ref[...]
    best = None
    besti = None
    for kt in range(NK):
        w = w_ref[pl.ds(kt * BK, BK), :]
        sqr = jnp.sum(w * w, axis=1, keepdims=True)
        cov = lax.dot_general(
            w, x,
            dimension_numbers=(((1,), (1,)), ((), ())),
            preferred_element_type=jnp.float32,
            precision=lax.Precision.DEFAULT,
        )
        scores = sqr - 2.0 * cov
        loc_min = jnp.min(scores, axis=0, keepdims=True)
        iota = lax.broadcasted_iota(jnp.int32, (BK, BM), 0) + kt * BK
        loc_arg = jnp.min(
            jnp.where(scores == loc_min, iota, K), axis=0, keepdims=True
        )
        if kt == 0:
            best, besti = loc_min, loc_arg
        else:
            upd = loc_min < best
            besti = jnp.where(upd, loc_arg, besti)
            best = jnp.where(upd, loc_min, best)
    idx_ref[...] = besti.reshape(1, 1, BM)


def _tc_argmin(x2, weight):
    return pl.pallas_call(
        _argmin_body,
        grid=(G,),
        in_specs=[
            pl.BlockSpec((BM, D), lambda m: (m, 0)),
            pl.BlockSpec((K, D), lambda m: (0, 0)),
        ],
        out_specs=pl.BlockSpec((1, 1, BM), lambda m: (m, 0, 0)),
        out_shape=jax.ShapeDtypeStruct((G, 1, BM), jnp.int32),
    )(x2, weight)


def _gather_body(idx_hbm, w_hbm, out_hbm, idx_v, rows_v, sem):
    wid = lax.axis_index("s") * NC + lax.axis_index("c")
    base = wid * ROWS_PER_W
    for c in range(NCH):
        off = base + c * GCH
        pltpu.sync_copy(idx_hbm.at[pl.ds(off, GCH)], idx_v)
        pltpu.async_copy(w_hbm.at[idx_v], rows_v, sem).wait()
        pltpu.sync_copy(rows_v, out_hbm.at[pl.ds(off, GCH)])


@functools.cache
def _sc_gather():
    return functools.partial(
        pl.kernel,
        out_type=jax.ShapeDtypeStruct((M, D), jnp.float32),
        mesh=plsc.VectorSubcoreMesh(core_axis_name="c", subcore_axis_name="s"),
        scratch_types=[
            pltpu.VMEM((GCH,), jnp.int32),
            pltpu.VMEM((GCH, D), jnp.float32),
            pltpu.SemaphoreType.DMA,
        ],
    )(_gather_body)


def kernel(input, weight):
    x2 = input.reshape(M, D)
    idx3 = _tc_argmin(x2, weight)
    idx_flat = idx3.reshape(M)
    vectors = _sc_gather()(idx_flat, weight).reshape(B, HW, D)
    indices = idx_flat.reshape(B, HW)
    return vectors, indices, vectors


# speedup vs baseline: 1.5782x; 1.1332x over previous
import functools

import jax
import jax.numpy as jnp
from jax import lax
from jax.experimental import pallas as pl
from jax.experimental.pallas import tpu as pltpu
from jax.experimental.pallas import tpu_sc as plsc

K = 8192
D = 256
B, HW = 32, 576
M = B * HW

BM = 512
BK = 512
G = M // BM
NK = K // BK

NC, NS = 2, 16
NW = NC * NS
ROWS_PER_W = M // NW
GCH = 96
NCH = ROWS_PER_W // GCH


def _argmin_body(x_ref, w_ref, idx_ref, sqr_ref):
    @pl.when(pl.program_id(0) == 0)
    def _():
        for kt in range(NK):
            w = w_ref[pl.ds(kt * BK, BK), :]
            sqr_ref[pl.ds(kt * BK, BK), :] = jnp.sum(w * w, axis=1, keepdims=True)

    x = x_ref[...] * -2.0
    best = None
    besti = None
    for kt in range(NK):
        w = w_ref[pl.ds(kt * BK, BK), :]
        cov = lax.dot_general(
            w, x,
            dimension_numbers=(((1,), (1,)), ((), ())),
            preferred_element_type=jnp.float32,
            precision=lax.Precision.DEFAULT,
        )
        scores = cov + sqr_ref[pl.ds(kt * BK, BK), :]
        loc_min = jnp.min(scores, axis=0, keepdims=True)
        iota = lax.broadcasted_iota(jnp.int32, (BK, BM), 0) + kt * BK
        loc_arg = jnp.min(
            jnp.where(scores == loc_min, iota, K), axis=0, keepdims=True
        )
        if kt == 0:
            best, besti = loc_min, loc_arg
        else:
            upd = loc_min < best
            besti = jnp.where(upd, loc_arg, besti)
            best = jnp.where(upd, loc_min, best)
    idx_ref[...] = besti.reshape(1, 1, BM)


def _tc_argmin(x2, weight):
    return pl.pallas_call(
        _argmin_body,
        grid=(G,),
        in_specs=[
            pl.BlockSpec((BM, D), lambda m: (m, 0)),
            pl.BlockSpec((K, D), lambda m: (0, 0)),
        ],
        out_specs=pl.BlockSpec((1, 1, BM), lambda m: (m, 0, 0)),
        out_shape=jax.ShapeDtypeStruct((G, 1, BM), jnp.int32),
        scratch_shapes=[pltpu.VMEM((K, 1), jnp.float32)],
    )(x2, weight)


def _gather_body(idx_hbm, w_hbm, out_hbm, idx_v, rows_v, sem):
    wid = lax.axis_index("s") * NC + lax.axis_index("c")
    base = wid * ROWS_PER_W
    for c in range(NCH):
        off = base + c * GCH
        pltpu.sync_copy(idx_hbm.at[pl.ds(off, GCH)], idx_v)
        pltpu.async_copy(w_hbm.at[idx_v], rows_v, sem).wait()
        pltpu.sync_copy(rows_v, out_hbm.at[pl.ds(off, GCH)])


@functools.cache
def _sc_gather():
    return functools.partial(
        pl.kernel,
        out_type=jax.ShapeDtypeStruct((M, D), jnp.float32),
        mesh=plsc.VectorSubcoreMesh(core_axis_name="c", subcore_axis_name="s"),
        scratch_types=[
            pltpu.VMEM((GCH,), jnp.int32),
            pltpu.VMEM((GCH, D), jnp.float32),
            pltpu.SemaphoreType.DMA,
        ],
    )(_gather_body)


def kernel(input, weight):
    x2 = input.reshape(M, D)
    idx3 = _tc_argmin(x2, weight)
    idx_flat = idx3.reshape(M)
    vectors = _sc_gather()(idx_flat, weight).reshape(B, HW, D)
    indices = idx_flat.reshape(B, HW)
    return vectors, indices, vectors


# speedup vs baseline: 1.7440x; 1.1051x over previous
import functools

import jax
import jax.numpy as jnp
from jax import lax
from jax.experimental import pallas as pl
from jax.experimental.pallas import tpu as pltpu
from jax.experimental.pallas import tpu_sc as plsc

K = 8192
D = 256
B, HW = 32, 576
M = B * HW

BM = 512
BK = 512
G = M // BM
NK = K // BK

NC, NS = 2, 16
NW = NC * NS
ROWS_PER_W = M // NW
GCH = 96
NCH = ROWS_PER_W // GCH


def _argmin_body(x_ref, w_ref, idx_ref, sqr_ref):
    @pl.when(pl.program_id(0) == 0)
    def _():
        for kt in range(NK):
            w = w_ref[pl.ds(kt * BK, BK), :]
            sqr_ref[pl.ds(kt * BK, BK), :] = jnp.sum(w * w, axis=1, keepdims=True)

    x = x_ref[...] * -2.0
    iota = lax.broadcasted_iota(jnp.int32, (BK, BM), 0).astype(jnp.float32)
    best = None
    besti = None
    for kt in range(NK):
        w = w_ref[pl.ds(kt * BK, BK), :]
        cov = lax.dot_general(
            w, x,
            dimension_numbers=(((1,), (1,)), ((), ())),
            preferred_element_type=jnp.float32,
            precision=lax.Precision.DEFAULT,
        )
        scores = cov + sqr_ref[pl.ds(kt * BK, BK), :]
        loc_min = jnp.min(scores, axis=0, keepdims=True)
        loc_arg = jnp.min(
            jnp.where(scores == loc_min, iota, float(K)), axis=0, keepdims=True
        ) + float(kt * BK)
        if kt == 0:
            best, besti = loc_min, loc_arg
        else:
            upd = loc_min < best
            besti = jnp.where(upd, loc_arg, besti)
            best = jnp.where(upd, loc_min, best)
    idx_ref[...] = besti.astype(jnp.int32).reshape(1, 1, BM)


def _tc_argmin(x2, weight):
    return pl.pallas_call(
        _argmin_body,
        grid=(G,),
        in_specs=[
            pl.BlockSpec((BM, D), lambda m: (m, 0)),
            pl.BlockSpec((K, D), lambda m: (0, 0)),
        ],
        out_specs=pl.BlockSpec((1, 1, BM), lambda m: (m, 0, 0)),
        out_shape=jax.ShapeDtypeStruct((G, 1, BM), jnp.int32),
        scratch_shapes=[pltpu.VMEM((K, 1), jnp.float32)],
    )(x2, weight)


def _gather_body(idx_hbm, w_hbm, out_hbm, idx_v, rows_v, sem):
    wid = lax.axis_index("s") * NC + lax.axis_index("c")
    base = wid * ROWS_PER_W
    for c in range(NCH):
        off = base + c * GCH
        pltpu.sync_copy(idx_hbm.at[pl.ds(off, GCH)], idx_v)
        pltpu.async_copy(w_hbm.at[idx_v], rows_v, sem).wait()
        pltpu.sync_copy(rows_v, out_hbm.at[pl.ds(off, GCH)])


@functools.cache
def _sc_gather():
    return functools.partial(
        pl.kernel,
        out_type=jax.ShapeDtypeStruct((M, D), jnp.float32),
        mesh=plsc.VectorSubcoreMesh(core_axis_name="c", subcore_axis_name="s"),
        scratch_types=[
            pltpu.VMEM((GCH,), jnp.int32),
            pltpu.VMEM((GCH, D), jnp.float32),
            pltpu.SemaphoreType.DMA,
        ],
    )(_gather_body)


def kernel(input, weight):
    x2 = input.reshape(M, D)
    idx3 = _tc_argmin(x2, weight)
    idx_flat = idx3.reshape(M)
    vectors = _sc_gather()(idx_flat, weight).reshape(B, HW, D)
    indices = idx_flat.reshape(B, HW)
    return vectors, indices, vectors
